# baseline (speedup 1.0000x reference)
import functools

import jax
import jax.numpy as jnp
from jax import lax
from jax.experimental import pallas as pl
from jax.experimental.pallas import tpu as pltpu
from jax.experimental.pallas import tpu_sc as plsc

_D = 64
_DP = 128
_NW = 32
_NBUF = 4
_QS = 16


def _body(nb, seq, idx_hbm, expr_hbm, table_hbm, out_hbm,
          idx_q0, idx_q1, expr_q0, expr_q1,
          g0, g1, g2, g3, gsem0, gsem1, gsem2, gsem3,
          ssem0, ssem1, ssem2, ssem3):
  nc = plsc.get_sparse_core_info().num_cores
  wid = lax.axis_index("s") * nc + lax.axis_index("c")
  base = wid * nb

  gbuf = (g0, g1, g2, g3)
  gsem = (gsem0, gsem1, gsem2, gsem3)
  ssem = (ssem0, ssem1, ssem2, ssem3)
  idx_q = (idx_q0, idx_q1)
  expr_q = (expr_q0, expr_q1)

  def stage(q, p):
    pltpu.sync_copy(idx_hbm.at[pl.ds(base + q * _QS, _QS)], idx_q[p])
    pltpu.sync_copy(expr_hbm.at[pl.ds(base + q * _QS, _QS)], expr_q[p])

  stage(0, 0)
  stage(1, 1)

  def gathers(j, b, iq):
    row = lax.rem(j, _QS)
    return (
        pltpu.make_async_copy(table_hbm.at[iq.at[row, pl.ds(0, 128)]],
                              gbuf[b].at[pl.ds(0, 128)], gsem[b]),
        pltpu.make_async_copy(table_hbm.at[iq.at[row, pl.ds(128, seq - 128)]],
                              gbuf[b].at[pl.ds(128, seq - 128)], gsem[b]),
    )

  def start_gathers(j, b):
    parity = lax.rem(lax.div(j, _QS), 2)

    @pl.when(parity == 0)
    def _():
      for c in gathers(j, b, idx_q0):
        c.start()

    @pl.when(parity == 1)
    def _():
      for c in gathers(j, b, idx_q1):
        c.start()

  def wait_gathers(j, b):
    for c in gathers(j, b, idx_q0):
      c.wait()

  def scatter(j, b):
    return pltpu.make_async_copy(gbuf[b].at[pl.ds(0, seq // 2)],
                                 out_hbm.at[base + j], ssem[b])

  start_gathers(0, 0)
  start_gathers(1, 1)

  n_full = seq // 16
  tail = seq - n_full * 16
  tail_base = seq - 16

  def scale_rows(j, b):
    row = lax.rem(j, _QS)
    par1 = lax.rem(lax.div(j, _QS), 2) == 1

    def expr_vec(sl):
      return jnp.where(par1, expr_q1[row, sl], expr_q0[row, sl])

    def group_body(g, _):
      ev = expr_vec(pl.ds(g * 16, 16))
      for r in range(16):
        e = ev[r]
        rr = g * 16 + r
        dr = g * 8 + r // 2
        for k in range(_D // 16):
          dl = pl.ds((r % 2) * _D + k * 16, 16)
          gbuf[b][dr, dl] = gbuf[b][rr, pl.ds(k * 16, 16)] * e
      return 0

    lax.fori_loop(0, n_full, group_body, 0)

    ev = expr_vec(pl.ds(tail_base, 16))
    for r in range(16 - tail, 16):
      e = ev[r]
      rr = tail_base + r
      dr = (tail_base + r) // 2
      for k in range(_D // 16):
        dl = pl.ds((r % 2) * _D + k * 16, 16)
        gbuf[b][dr, dl] = gbuf[b][rr, pl.ds(k * 16, 16)] * e

  nq = nb // _QS

  def outer(io):
    @pl.when(jnp.logical_and(lax.rem(io, 2 * _QS) == _QS,
                             io < (nq - 1) * _QS))
    def _():
      stage(lax.div(io, _QS) + 1, 0)

    @pl.when(jnp.logical_and(
        jnp.logical_and(lax.rem(io, 2 * _QS) == 0, io >= 2 * _QS),
        io < (nq - 1) * _QS))
    def _():
      stage(lax.div(io, _QS) + 1, 1)

    for b in range(_NBUF):
      i = io + b
      wait_gathers(i, b)
      scale_rows(i, b)
      scatter(i, b).start()

      nxt = (b + 2) % _NBUF

      @pl.when(i >= 2)
      def _():
        scatter(i - 2, nxt).wait()

      @pl.when(i + 2 < nb)
      def _():
        start_gathers(i + 2, nxt)

  pl.loop(0, nb, step=_NBUF)(outer)

  scatter(nb - 2, (nb - 2) % _NBUF).wait()
  scatter(nb - 1, (nb - 1) % _NBUF).wait()


def _relayout_body(x_ref, o_ref):
  x = x_ref[...]
  for r in range(x.shape[1]):
    o_ref[:, 2 * r, :] = x[:, r, :_D]
    o_ref[:, 2 * r + 1, :] = x[:, r, _D:]


def kernel(gene_indices, expression_values, embedding_table):
  bsz, seq = gene_indices.shape
  assert bsz % _NW == 0 and seq == 200
  nb = bsz // _NW

  table_pad = jnp.pad(embedding_table, ((0, 0), (0, _DP - _D)))

  mesh = plsc.VectorSubcoreMesh(core_axis_name="c", subcore_axis_name="s")
  inter = pl.kernel(
      functools.partial(_body, nb, seq),
      out_type=jax.ShapeDtypeStruct((bsz, seq // 2, 2 * _D), jnp.float32),
      mesh=mesh,
      compiler_params=pltpu.CompilerParams(use_tc_tiling_on_sc=True),
      scratch_types=[
          pltpu.VMEM((_QS, seq), jnp.int32),
          pltpu.VMEM((_QS, seq), jnp.int32),
          pltpu.VMEM((_QS, seq), jnp.float32),
          pltpu.VMEM((_QS, seq), jnp.float32),
      ] + [pltpu.VMEM((seq, _DP), jnp.float32)] * _NBUF
        + [pltpu.SemaphoreType.DMA] * (2 * _NBUF),
  )(gene_indices.astype(jnp.int32), expression_values, table_pad)

  bb = 64
  out = pl.pallas_call(
      _relayout_body,
      grid=(bsz // bb,),
      in_specs=[pl.BlockSpec((bb, seq // 2, 2 * _D), lambda i: (i, 0, 0))],
      out_specs=pl.BlockSpec((bb, seq, _D), lambda i: (i, 0, 0)),
      out_shape=jax.ShapeDtypeStruct((bsz, seq, _D), jnp.float32),
  )(inter)
  return out


# speedup vs baseline: 1.5361x; 1.5361x over previous
import functools

import jax
import jax.numpy as jnp
from jax import lax
from jax.experimental import pallas as pl
from jax.experimental.pallas import tpu as pltpu
from jax.experimental.pallas import tpu_sc as plsc

_D = 64
_NW = 32
_NBUF = 2


def _body(nb, seq, idx_hbm, table_hbm, out_hbm, idx_v,
          g0, g1, s0, s1, gsem0, gsem1, ssem0, ssem1):
  nc = plsc.get_sparse_core_info().num_cores
  wid = lax.axis_index("s") * nc + lax.axis_index("c")
  base = wid * nb

  gbuf = (g0, g1)
  sbuf = (s0, s1)
  gsem = (gsem0, gsem1)
  ssem = (ssem0, ssem1)

  pltpu.sync_copy(idx_hbm.at[pl.ds(base, nb)], idx_v)

  def gathers(i, b):
    return (
        pltpu.make_async_copy(table_hbm.at[idx_v.at[i, pl.ds(0, 128)]],
                              gbuf[b].at[pl.ds(0, 128)], gsem[b]),
        pltpu.make_async_copy(table_hbm.at[idx_v.at[i, pl.ds(128, seq - 128)]],
                              gbuf[b].at[pl.ds(128, seq - 128)], gsem[b]),
    )

  def scatter(i, b):
    return pltpu.make_async_copy(sbuf[b], out_hbm.at[base + i], ssem[b])

  def start_gathers(i, b):
    for c in gathers(i, b):
      c.start()

  def wait_gathers(i, b):
    for c in gathers(i, b):
      c.wait()

  for b in range(_NBUF):
    start_gathers(b, b)

  n_full = seq // 16
  tail = seq - n_full * 16
  tail_base = seq - 16

  def pack_rows(b):
    def group_body(g, _):
      for r in range(16):
        rr = g * 16 + r
        dr = g * 8 + r // 2
        for k in range(_D // 16):
          dl = pl.ds((r % 2) * _D + k * 16, 16)
          sbuf[b][dr, dl] = gbuf[b][rr, pl.ds(k * 16, 16)]
      return 0

    lax.fori_loop(0, n_full, group_body, 0)

    for r in range(16 - tail, 16):
      rr = tail_base + r
      dr = rr // 2
      for k in range(_D // 16):
        dl = pl.ds((r % 2) * _D + k * 16, 16)
        sbuf[b][dr, dl] = gbuf[b][rr, pl.ds(k * 16, 16)]

  def outer(io):
    for b in range(_NBUF):
      i = io + b
      wait_gathers(i, b)

      @pl.when(i >= _NBUF)
      def _():
        scatter(i - _NBUF, b).wait()

      pack_rows(b)
      scatter(i, b).start()

      @pl.when(i + _NBUF < nb)
      def _():
        start_gathers(i + _NBUF, b)

  pl.loop(0, nb, step=_NBUF)(outer)

  for b in range(_NBUF):
    scatter(nb - _NBUF + b, b).wait()


def kernel(gene_indices, expression_values, embedding_table):
  bsz, seq = gene_indices.shape
  assert bsz % _NW == 0 and seq == 200
  nb = bsz // _NW

  mesh = plsc.VectorSubcoreMesh(core_axis_name="c", subcore_axis_name="s")
  inter = pl.kernel(
      functools.partial(_body, nb, seq),
      out_type=jax.ShapeDtypeStruct((bsz, seq // 2, 2 * _D), jnp.float32),
      mesh=mesh,
      compiler_params=pltpu.CompilerParams(use_tc_tiling_on_sc=False),
      scratch_types=[
          pltpu.VMEM((nb, seq), jnp.int32),
      ] + [pltpu.VMEM((seq, _D), jnp.float32)] * _NBUF
        + [pltpu.VMEM((seq // 2, 2 * _D), jnp.float32)] * _NBUF
        + [pltpu.SemaphoreType.DMA] * (2 * _NBUF),
  )(gene_indices.astype(jnp.int32), embedding_table)

  return inter.reshape(bsz, seq, _D) * expression_values[..., None]
